# baseline (speedup 1.0000x reference)
import jax
import jax.numpy as jnp
from jax import lax
from jax.experimental import pallas as pl
from jax.experimental.pallas import tpu as pltpu
from jax.experimental.pallas import tpu_sc as plsc

NN = 10000
NPAD = 10240
EE = 20000
EPAD = 20480
PAD_BIN = 10200
FF = 128
IN_F = 512
NC, NS, L = 2, 16, 16
NW = NC * NS
EPW = EPAD // NW
CH = 64
NCHUNK = EPW // CH
ZROWS = NPAD // NS
HB = NPAD // 2

SRC_T = [0, 1, 0, 1, 0, 2, 0, 3, 1, 2]
SRC_GROUPS = [[0, 2, 4, 6], [1, 3, 8], [5, 9], [7]]
DST_GROUPS = [[0, 3, 5, 7], [1, 2, 9], [4, 8], [6]]
DST_ORDER = [r for g in DST_GROUPS for r in g]
SLOT = {}
for _g in SRC_GROUPS:
    for _pos, _r in enumerate(_g):
        SLOT[_r] = _pos


def _sc_mesh():
    return plsc.VectorSubcoreMesh(
        core_axis_name="c", subcore_axis_name="s",
        num_cores=NC, num_subcores=NS)


def _deg_body(srcc_ref, dstf_ref, cnt_ref, idx_buf, hl, red):
    wid = lax.axis_index("s") * NC + lax.axis_index("c")

    @pl.when(wid < 20)
    def _():
        @pl.when(wid < 10)
        def _():
            pltpu.sync_copy(srcc_ref.at[wid], idx_buf)

        @pl.when(wid >= 10)
        def _():
            pltpu.sync_copy(dstf_ref.at[wid - 10], idx_buf)

        lanes = lax.iota(jnp.int32, L)
        ones = jnp.ones((L,), jnp.float32)
        zeros = jnp.zeros((L,), jnp.float32)
        for half in range(2):
            base = half * HB

            def _zero(i, carry):
                for row in range(L):
                    hl[pl.ds(row * HB + i * L, L)] = zeros
                return carry
            lax.fori_loop(0, HB // L, _zero, 0)

            def _cnt(i, carry):
                ix = idx_buf[pl.ds(i * L, L)] - base
                m = (ix >= 0) & (ix < HB)
                plsc.addupdate_scatter(hl, [lanes * HB + ix], ones, mask=m)
                return carry
            lax.fori_loop(0, EPAD // L, _cnt, 0)

            def _red(i, carry):
                s = hl[pl.ds(i * L, L)]
                for row in range(1, L):
                    s = s + hl[pl.ds(row * HB + i * L, L)]
                red[pl.ds(i * L, L)] = s
                return carry
            lax.fori_loop(0, HB // L, _red, 0)
            pltpu.sync_copy(red, cnt_ref.at[wid, pl.ds(base, HB)])


def _rsqrt_body(cnt_ref, r_ref):
    r_ref[...] = lax.rsqrt(jnp.maximum(cnt_ref[...], 1.0))


def _mm_body(x_ref, w_ref, o_ref):
    o_ref[0] = jnp.dot(x_ref[...], w_ref[0], preferred_element_type=jnp.float32)


def _matmul(x, wstack):
    k = wstack.shape[0]
    bm = 2048
    return pl.pallas_call(
        _mm_body,
        grid=(pl.cdiv(NN, bm), k),
        in_specs=[
            pl.BlockSpec((bm, IN_F), lambda i, r: (i, 0)),
            pl.BlockSpec((1, IN_F, FF), lambda i, r: (r, 0, 0)),
        ],
        out_specs=pl.BlockSpec((1, bm, FF), lambda i, r: (r, i, 0)),
        out_shape=jax.ShapeDtypeStruct((k, NN, FF), jnp.float32),
    )(x, wstack)


CORE_TYPES = [[0, 3], [1, 2]]
EPW2 = EPAD // NS
NCHUNK2 = EPW2 // CH


def _agg_body(h0, h1, h2, h3, srcg_ref, dstg_ref, rmat_ref, zrows_ref,
              part_ref, acc, zb, src_b, dst_b, rout, rin, wb, rows,
              sem0, sem1):
    cid = lax.axis_index("c")
    sid = lax.axis_index("s")
    htabs = [h0, h1, h2, h3]
    sems = [sem0, sem1]
    zh = zrows_ref.shape[0]

    pltpu.sync_copy(zrows_ref, zb)

    def _do_type(t):
        for j in range(ZROWS // zh):
            pltpu.sync_copy(zb, acc.at[pl.ds(sid * ZROWS + j * zh, zh)])
        plsc.subcore_barrier()

        for r in DST_GROUPS[t]:
            htab = htabs[SRC_T[r]]
            base = SLOT[r] * NN
            pltpu.sync_copy(rmat_ref.at[r], rout)
            pltpu.sync_copy(rmat_ref.at[10 + r], rin)
            pltpu.sync_copy(srcg_ref.at[r, sid], src_b)
            pltpu.sync_copy(dstg_ref.at[r, sid], dst_b)

            def _gath(c, buf):
                return pltpu.async_copy(
                    htab.at[src_b.at[c]], rows.at[buf], sems[buf])

            def _chunk(c, buf):
                pltpu.make_async_copy(
                    htab.at[src_b.at[0]], rows.at[buf], sems[buf]).wait()

                def _wbody(g, carry):
                    si = src_b[c, pl.ds(g * L, L)] - base
                    di = dst_b[c, pl.ds(g * L, L)]
                    wb[pl.ds(g * L, L)] = (plsc.load_gather(rout, [si])
                                           * plsc.load_gather(rin, [di]))
                    return carry
                lax.fori_loop(0, CH // L, _wbody, 0)

                def _ebody(e, carry):
                    ws = plsc.load_gather(wb, [jnp.broadcast_to(e, (L,))])
                    for j in range(FF // L):
                        sl = pl.ds(j * L, L)
                        rows[buf, e, sl] = rows[buf, e, sl] * ws
                    return carry
                lax.fori_loop(0, CH, _ebody, 0)

                pltpu.sync_copy(rows.at[buf], acc.at[dst_b.at[c]], add=True)

            _gath(0, 0)

            def _pair(p, carry):
                c0 = p * 2
                for par in range(2):
                    c = c0 + par

                    @pl.when(c + 1 < NCHUNK2)
                    def _():
                        _gath(c + 1, (par + 1) % 2)
                    _chunk(c, par)
                return carry
            lax.fori_loop(0, NCHUNK2 // 2, _pair, 0)

        plsc.subcore_barrier()
        pltpu.sync_copy(acc.at[pl.ds(sid * ZROWS, ZROWS)],
                        part_ref.at[t, pl.ds(sid * ZROWS, ZROWS)])
        plsc.subcore_barrier()

    @pl.when(cid == 0)
    def _():
        for t in CORE_TYPES[0]:
            _do_type(t)

    @pl.when(cid == 1)
    def _():
        for t in CORE_TYPES[1]:
            _do_type(t)


def _fin_body(p_ref, b_ref, o_ref):
    s = p_ref[...] + jnp.sum(b_ref[...], axis=0, keepdims=True)
    o_ref[...] = jnp.maximum(s, 0.0)


def kernel(x_drug, x_protein, x_disease, x_sideeffect,
           ei_dd, ei_pp, ei_dp, ei_pd, ei_ddi, ei_did, ei_dse, ei_sed,
           ei_pdi, ei_dip,
           W_dd, b_dd, W_pp, b_pp, W_dp, b_dp, W_pd, b_pd, W_ddi, b_ddi,
           W_did, b_did, W_dse, b_dse, W_sed, b_sed, W_pdi, b_pdi,
           W_dip, b_dip):
    xs = [x_drug, x_protein, x_disease, x_sideeffect]
    eis = [ei_dd, ei_pp, ei_dp, ei_pd, ei_ddi, ei_did, ei_dse, ei_sed,
           ei_pdi, ei_dip]
    Ws = [W_dd, W_pp, W_dp, W_pd, W_ddi, W_did, W_dse, W_sed, W_pdi, W_dip]
    bs = [b_dd, b_pp, b_dp, b_pd, b_ddi, b_did, b_dse, b_sed, b_pdi, b_dip]

    npd = EPAD - EE
    srcall = jnp.stack([eis[r][0] for r in range(10)])
    dstall = jnp.stack([eis[r][1] for r in range(10)])
    slots = jnp.array([SLOT[r] * NN for r in range(10)], jnp.int32)
    srcc = jnp.pad(srcall, ((0, 0), (0, npd)), constant_values=PAD_BIN)
    dstf = jnp.pad(dstall, ((0, 0), (0, npd)), constant_values=PAD_BIN)
    srcg = jnp.pad(srcall, ((0, 0), (0, npd))) + slots[:, None]
    srcg4 = srcg.reshape(10, NS, EPW2 // CH, CH)
    dstg4 = dstf.reshape(10, NS, EPW2 // CH, CH)

    cnt = pl.kernel(
        _deg_body,
        out_type=jax.ShapeDtypeStruct((20, NPAD), jnp.float32),
        mesh=_sc_mesh(),
        compiler_params=pltpu.CompilerParams(needs_layout_passes=False),
        scratch_types=[
            pltpu.VMEM((EPAD,), jnp.int32),
            pltpu.VMEM((L * HB,), jnp.float32),
            pltpu.VMEM((HB,), jnp.float32),
        ],
    )(srcc, dstf)

    rmat = pl.pallas_call(
        _rsqrt_body,
        out_shape=jax.ShapeDtypeStruct((20, NPAD), jnp.float32),
    )(cnt)

    htabs = []
    for tt in range(4):
        wstack = jnp.stack([Ws[r] for r in SRC_GROUPS[tt]])
        h = _matmul(xs[tt], wstack)
        htabs.append(h.reshape(len(SRC_GROUPS[tt]) * NN, FF))

    zrows = jnp.zeros((16, FF), jnp.float32)
    part = pl.kernel(
        _agg_body,
        out_type=jax.ShapeDtypeStruct((4, NPAD, FF), jnp.float32),
        mesh=_sc_mesh(),
        compiler_params=pltpu.CompilerParams(needs_layout_passes=False),
        scratch_types=[
            pltpu.VMEM_SHARED((NPAD, FF), jnp.float32),
            pltpu.VMEM((16, FF), jnp.float32),
            pltpu.VMEM((NCHUNK2, CH), jnp.int32),
            pltpu.VMEM((NCHUNK2, CH), jnp.int32),
            pltpu.VMEM((NPAD,), jnp.float32),
            pltpu.VMEM((NPAD,), jnp.float32),
            pltpu.VMEM((CH,), jnp.float32),
            pltpu.VMEM((2, CH, FF), jnp.float32),
            pltpu.SemaphoreType.DMA,
            pltpu.SemaphoreType.DMA,
        ],
    )(htabs[0], htabs[1], htabs[2], htabs[3], srcg4, dstg4, rmat, zrows)

    bmd = 1024
    outs = []
    for t in range(4):
        bsum = jnp.stack([bs[r] for r in DST_GROUPS[t]])
        out_t = pl.pallas_call(
            _fin_body,
            grid=(NPAD // bmd,),
            in_specs=[
                pl.BlockSpec((bmd, FF), lambda i: (i, 0)),
                pl.BlockSpec((len(DST_GROUPS[t]), FF), lambda i: (0, 0)),
            ],
            out_specs=pl.BlockSpec((bmd, FF), lambda i: (i, 0)),
            out_shape=jax.ShapeDtypeStruct((NN, FF), jnp.float32),
        )(part[t], bsum)
        outs.append(out_t)

    return tuple(outs)


# speedup vs baseline: 1.0215x; 1.0215x over previous
import jax
import jax.numpy as jnp
from jax import lax
from jax.experimental import pallas as pl
from jax.experimental.pallas import tpu as pltpu
from jax.experimental.pallas import tpu_sc as plsc

NN = 10000
NPAD = 10240
EE = 20000
EPAD = 20480
PAD_BIN = 10200
FF = 128
IN_F = 512
NC, NS, L = 2, 16, 16
NW = NC * NS
EPW = EPAD // NW
CH = 64
NCHUNK = EPW // CH
ZROWS = NPAD // NS
HB = NPAD // 2

SRC_T = [0, 1, 0, 1, 0, 2, 0, 3, 1, 2]
SRC_GROUPS = [[0, 2, 4, 6], [1, 3, 8], [5, 9], [7]]
DST_GROUPS = [[0, 3, 5, 7], [1, 2, 9], [4, 8], [6]]
DST_ORDER = [r for g in DST_GROUPS for r in g]
SLOT = {}
for _g in SRC_GROUPS:
    for _pos, _r in enumerate(_g):
        SLOT[_r] = _pos


def _sc_mesh():
    return plsc.VectorSubcoreMesh(
        core_axis_name="c", subcore_axis_name="s",
        num_cores=NC, num_subcores=NS)


def _deg_body(srcc_ref, dstf_ref, cnt_ref, idx_buf, hl, red):
    wid = lax.axis_index("s") * NC + lax.axis_index("c")

    @pl.when(wid < 20)
    def _():
        @pl.when(wid < 10)
        def _():
            pltpu.sync_copy(srcc_ref.at[wid], idx_buf)

        @pl.when(wid >= 10)
        def _():
            pltpu.sync_copy(dstf_ref.at[wid - 10], idx_buf)

        lanes = lax.iota(jnp.int32, L)
        ones = jnp.ones((L,), jnp.float32)
        zeros = jnp.zeros((L,), jnp.float32)
        for half in range(2):
            base = half * HB

            def _zero(i, carry):
                for row in range(L):
                    hl[pl.ds(row * HB + i * L, L)] = zeros
                return carry
            lax.fori_loop(0, HB // L, _zero, 0)

            def _cnt(i, carry):
                ix = idx_buf[pl.ds(i * L, L)] - base
                m = (ix >= 0) & (ix < HB)
                plsc.addupdate_scatter(hl, [lanes * HB + ix], ones, mask=m)
                return carry
            lax.fori_loop(0, EPAD // L, _cnt, 0)

            def _red(i, carry):
                s = hl[pl.ds(i * L, L)]
                for row in range(1, L):
                    s = s + hl[pl.ds(row * HB + i * L, L)]
                red[pl.ds(i * L, L)] = s
                return carry
            lax.fori_loop(0, HB // L, _red, 0)
            pltpu.sync_copy(red, cnt_ref.at[wid, pl.ds(base, HB)])


def _rsqrt_body(cnt_ref, r_ref):
    r_ref[...] = lax.rsqrt(jnp.maximum(cnt_ref[...], 1.0))


def _mm_body(x_ref, w_ref, o_ref):
    o_ref[0] = jnp.dot(x_ref[...], w_ref[0], preferred_element_type=jnp.float32)


def _matmul(x, wstack):
    k = wstack.shape[0]
    bm = 2048
    return pl.pallas_call(
        _mm_body,
        grid=(pl.cdiv(NN, bm), k),
        in_specs=[
            pl.BlockSpec((bm, IN_F), lambda i, r: (i, 0)),
            pl.BlockSpec((1, IN_F, FF), lambda i, r: (r, 0, 0)),
        ],
        out_specs=pl.BlockSpec((1, bm, FF), lambda i, r: (r, i, 0)),
        out_shape=jax.ShapeDtypeStruct((k, NN, FF), jnp.float32),
    )(x, wstack)


CORE_TYPES = [[0, 3], [1, 2]]
EPW2 = EPAD // NS
NCHUNK2 = EPW2 // CH


def _agg_body(h0, h1, h2, h3, srcg_ref, dstg_ref, rmat_ref, zrows_ref,
              part_ref, acc, zb, src_b, dst_b, rout, rin, wb, rows,
              sem0, sem1):
    cid = lax.axis_index("c")
    sid = lax.axis_index("s")
    htabs = [h0, h1, h2, h3]
    sems = [sem0, sem1]
    zh = zrows_ref.shape[0]

    pltpu.sync_copy(zrows_ref, zb)

    def _do_type(t):
        for j in range(ZROWS // zh):
            pltpu.sync_copy(zb, acc.at[pl.ds(sid * ZROWS + j * zh, zh)])
        plsc.subcore_barrier()

        for r in DST_GROUPS[t]:
            htab = htabs[SRC_T[r]]
            base = SLOT[r] * NN
            pltpu.sync_copy(rmat_ref.at[r], rout)
            pltpu.sync_copy(rmat_ref.at[10 + r], rin)
            pltpu.sync_copy(srcg_ref.at[r, sid], src_b)
            pltpu.sync_copy(dstg_ref.at[r, sid], dst_b)

            def _gath(c, buf):
                return pltpu.async_copy(
                    htab.at[src_b.at[c]], rows.at[buf], sems[buf])

            def _chunk(c, buf):
                pltpu.make_async_copy(
                    htab.at[src_b.at[0]], rows.at[buf], sems[buf]).wait()

                def _wbody(g, carry):
                    si = src_b[c, pl.ds(g * L, L)] - base
                    di = dst_b[c, pl.ds(g * L, L)]
                    wb[pl.ds(g * L, L)] = (plsc.load_gather(rout, [si])
                                           * plsc.load_gather(rin, [di]))
                    return carry
                lax.fori_loop(0, CH // L, _wbody, 0)

                def _ebody(e, carry):
                    ws = plsc.load_gather(wb, [jnp.broadcast_to(e, (L,))])
                    for j in range(FF // L):
                        sl = pl.ds(j * L, L)
                        rows[buf, e, sl] = rows[buf, e, sl] * ws
                    return carry
                lax.fori_loop(0, CH, _ebody, 0)

                pltpu.sync_copy(rows.at[buf], acc.at[dst_b.at[c]], add=True)

            _gath(0, 0)

            def _pair(p, carry):
                c0 = p * 2
                for par in range(2):
                    c = c0 + par

                    @pl.when(c + 1 < NCHUNK2)
                    def _():
                        _gath(c + 1, (par + 1) % 2)
                    _chunk(c, par)
                return carry
            lax.fori_loop(0, NCHUNK2 // 2, _pair, 0)

        plsc.subcore_barrier()
        pltpu.sync_copy(acc.at[pl.ds(sid * ZROWS, ZROWS)],
                        part_ref.at[t, pl.ds(sid * ZROWS, ZROWS)])
        plsc.subcore_barrier()

    @pl.when(cid == 0)
    def _():
        for t in CORE_TYPES[0]:
            _do_type(t)

    @pl.when(cid == 1)
    def _():
        for t in CORE_TYPES[1]:
            _do_type(t)


def _fin_body(p_ref, b_ref, o_ref):
    s = p_ref[0] + jnp.sum(b_ref[0], axis=0, keepdims=True)
    o_ref[0] = jnp.maximum(s, 0.0)


def kernel(x_drug, x_protein, x_disease, x_sideeffect,
           ei_dd, ei_pp, ei_dp, ei_pd, ei_ddi, ei_did, ei_dse, ei_sed,
           ei_pdi, ei_dip,
           W_dd, b_dd, W_pp, b_pp, W_dp, b_dp, W_pd, b_pd, W_ddi, b_ddi,
           W_did, b_did, W_dse, b_dse, W_sed, b_sed, W_pdi, b_pdi,
           W_dip, b_dip):
    xs = [x_drug, x_protein, x_disease, x_sideeffect]
    eis = [ei_dd, ei_pp, ei_dp, ei_pd, ei_ddi, ei_did, ei_dse, ei_sed,
           ei_pdi, ei_dip]
    Ws = [W_dd, W_pp, W_dp, W_pd, W_ddi, W_did, W_dse, W_sed, W_pdi, W_dip]
    bs = [b_dd, b_pp, b_dp, b_pd, b_ddi, b_did, b_dse, b_sed, b_pdi, b_dip]

    npd = EPAD - EE
    srcall = jnp.stack([eis[r][0] for r in range(10)])
    dstall = jnp.stack([eis[r][1] for r in range(10)])
    slots = jnp.array([SLOT[r] * NN for r in range(10)], jnp.int32)
    srcc = jnp.pad(srcall, ((0, 0), (0, npd)), constant_values=PAD_BIN)
    dstf = jnp.pad(dstall, ((0, 0), (0, npd)), constant_values=PAD_BIN)
    srcg = jnp.pad(srcall, ((0, 0), (0, npd))) + slots[:, None]
    srcg4 = srcg.reshape(10, NS, EPW2 // CH, CH)
    dstg4 = dstf.reshape(10, NS, EPW2 // CH, CH)

    cnt = pl.kernel(
        _deg_body,
        out_type=jax.ShapeDtypeStruct((20, NPAD), jnp.float32),
        mesh=_sc_mesh(),
        compiler_params=pltpu.CompilerParams(needs_layout_passes=False),
        scratch_types=[
            pltpu.VMEM((EPAD,), jnp.int32),
            pltpu.VMEM((L * HB,), jnp.float32),
            pltpu.VMEM((HB,), jnp.float32),
        ],
    )(srcc, dstf)

    rmat = pl.pallas_call(
        _rsqrt_body,
        out_shape=jax.ShapeDtypeStruct((20, NPAD), jnp.float32),
    )(cnt)

    htabs = []
    for tt in range(4):
        wstack = jnp.stack([Ws[r] for r in SRC_GROUPS[tt]])
        h = _matmul(xs[tt], wstack)
        htabs.append(h.reshape(len(SRC_GROUPS[tt]) * NN, FF))

    zrows = jnp.zeros((16, FF), jnp.float32)
    part = pl.kernel(
        _agg_body,
        out_type=jax.ShapeDtypeStruct((4, NPAD, FF), jnp.float32),
        mesh=_sc_mesh(),
        compiler_params=pltpu.CompilerParams(needs_layout_passes=False),
        scratch_types=[
            pltpu.VMEM_SHARED((NPAD, FF), jnp.float32),
            pltpu.VMEM((16, FF), jnp.float32),
            pltpu.VMEM((NCHUNK2, CH), jnp.int32),
            pltpu.VMEM((NCHUNK2, CH), jnp.int32),
            pltpu.VMEM((NPAD,), jnp.float32),
            pltpu.VMEM((NPAD,), jnp.float32),
            pltpu.VMEM((CH,), jnp.float32),
            pltpu.VMEM((2, CH, FF), jnp.float32),
            pltpu.SemaphoreType.DMA,
            pltpu.SemaphoreType.DMA,
        ],
    )(htabs[0], htabs[1], htabs[2], htabs[3], srcg4, dstg4, rmat, zrows)

    brows = []
    for t in range(4):
        bsum = [bs[r] for r in DST_GROUPS[t]]
        bsum = bsum + [jnp.zeros((FF,), jnp.float32)] * (4 - len(bsum))
        brows.append(jnp.stack(bsum))
    bstack = jnp.stack(brows)

    bmd = 1024
    out = pl.pallas_call(
        _fin_body,
        grid=(4, NPAD // bmd),
        in_specs=[
            pl.BlockSpec((1, bmd, FF), lambda t, i: (t, i, 0)),
            pl.BlockSpec((1, 4, FF), lambda t, i: (t, 0, 0)),
        ],
        out_specs=pl.BlockSpec((1, bmd, FF), lambda t, i: (t, i, 0)),
        out_shape=jax.ShapeDtypeStruct((4, NN, FF), jnp.float32),
    )(part, bstack)

    return (out[0], out[1], out[2], out[3])
